# Initial kernel scaffold; baseline (speedup 1.0000x reference)
import jax
import jax.numpy as jnp
from jax.experimental import pallas as pl


def kernel(logit, target, class_for_batch):
    raise NotImplementedError("write your pallas kernel here")


# speedup vs baseline: 99.0681x; 99.0681x over previous
import functools

import jax
import jax.numpy as jnp
from jax.experimental import pallas as pl
from jax.experimental.pallas import tpu as pltpu

_SMOOTH = 1e-05


def _ce_body(lref, tref, aref, loss_ref, cnt_ref, *, num_class):
    b = pl.program_id(0)
    j = pl.program_id(1)
    t = tref[0]
    sel = jnp.zeros(t.shape, jnp.float32)
    w = jnp.zeros(t.shape, jnp.float32)
    for c in range(num_class):
        m = t == c
        sel = jnp.where(m, lref[0, c], sel)
        w = w + m.astype(jnp.float32) * aref[0, c]
    lp = jnp.log(jnp.clip(sel, _SMOOTH, 1.0))
    contrib = w * (lp + _SMOOTH)
    part = jnp.sum(contrib, axis=0, keepdims=True)
    cnt = jnp.sum((t > 0).astype(jnp.float32), axis=0, keepdims=True)

    @pl.when((b == 0) & (j == 0))
    def _init():
        loss_ref[...] = jnp.zeros_like(loss_ref)
        cnt_ref[...] = jnp.zeros_like(cnt_ref)

    loss_ref[...] += part
    cnt_ref[...] += cnt


def kernel(logit, target, class_for_batch):
    B, C, H, W = logit.shape
    HW = H * W
    n = B * HW
    present = (jnp.arange(C)[:, None] == class_for_batch[None, :]).any(axis=1)
    alpha = jnp.where(present, 1.0, 0.0).astype(jnp.float32)
    alpha = alpha.at[0].set(0.0)
    aref = jnp.zeros((1, 128), jnp.float32).at[0, :C].set(alpha)

    lanes = 128
    rows = HW // lanes
    R = 512
    nj = rows // R
    lg = logit.reshape(B, C, rows, lanes)
    tg = target.reshape(B, rows, lanes)

    grid = (B, nj)
    loss_part, cnt_part = pl.pallas_call(
        functools.partial(_ce_body, num_class=C),
        grid=grid,
        in_specs=[
            pl.BlockSpec((1, C, R, lanes), lambda b, j: (b, 0, j, 0)),
            pl.BlockSpec((1, R, lanes), lambda b, j: (b, j, 0)),
            pl.BlockSpec((1, lanes), lambda b, j: (0, 0)),
        ],
        out_specs=[
            pl.BlockSpec((1, lanes), lambda b, j: (0, 0)),
            pl.BlockSpec((1, lanes), lambda b, j: (0, 0)),
        ],
        out_shape=[
            jax.ShapeDtypeStruct((1, lanes), jnp.float32),
            jax.ShapeDtypeStruct((1, lanes), jnp.float32),
        ],
    )(lg, tg, aref)

    s = -jnp.sum(loss_part)
    pos = jnp.sum(cnt_part)
    return jnp.where(pos > 0, s / pos, s / jnp.float32(n))
